# baseline (speedup 1.0000x reference)
import functools

import jax
import jax.numpy as jnp
from jax import lax
from jax.experimental import pallas as pl
from jax.experimental.pallas import tpu as pltpu
from jax.experimental.pallas import tpu_sc as plsc

B = 16384
D = 64
NC = 2
NS = 16
NW = NC * NS
BW = B // NW
NCHUNK = 4
CHUNK = BW // NCHUNK
GROUPS = BW // 16
D_PAD = 65


def _sc_body(rs_hbm, ds_hbm, rw_hbm, dw_hbm, out_hbm,
             rs_v, ds_v, r_rows, d_rows, out_v, r_spm, d_spm, sem):
    wid = lax.axis_index("s") * NC + lax.axis_index("c")
    base = wid * BW

    icp0 = pltpu.async_copy(rs_hbm.at[pl.ds(base, BW)], rs_v, sem.at[8])
    icp1 = pltpu.async_copy(ds_hbm.at[pl.ds(base, BW)], ds_v, sem.at[9])

    sid = lax.axis_index("s")
    row0 = sid * 64

    @pl.when(sid < 15)
    def _stage_main():
        pltpu.sync_copy(rw_hbm.at[pl.ds(row0, 64)], r_spm.at[pl.ds(row0, 64)])
        pltpu.sync_copy(dw_hbm.at[pl.ds(row0, 64)], d_spm.at[pl.ds(row0, 64)])

    @pl.when(sid == 15)
    def _stage_tail():
        pltpu.sync_copy(rw_hbm.at[pl.ds(960, 40)], r_spm.at[pl.ds(960, 40)])
        pltpu.sync_copy(dw_hbm.at[pl.ds(960, 40)], d_spm.at[pl.ds(960, 40)])

    plsc.subcore_barrier()

    icp0.wait()
    icp1.wait()

    NPIPE = 4
    HALF = BW // NPIPE
    cps = []
    for h in range(NPIPE):
        sl = pl.ds(h * HALF, HALF)
        cps.append((
            pltpu.async_copy(r_spm.at[rs_v.at[sl]], r_rows.at[sl],
                             sem.at[2 * h]),
            pltpu.async_copy(d_spm.at[ds_v.at[sl]], d_rows.at[sl],
                             sem.at[2 * h + 1]),
        ))

    lane = lax.iota(jnp.int32, 16)
    last_lane = lane == 15

    for h in range(NPIPE):
        cps[h][0].wait()
        cps[h][1].wait()

        @plsc.parallel_loop(h * HALF, (h + 1) * HALF, 4, unroll=2)
        def block_body(b0):
            for u in range(4):
                b = b0 + u
                acc = (r_rows[b, pl.ds(0, 16)] * d_rows[b, pl.ds(0, 16)] +
                       r_rows[b, pl.ds(16, 16)] * d_rows[b, pl.ds(16, 16)]) + (
                       r_rows[b, pl.ds(32, 16)] * d_rows[b, pl.ds(32, 16)] +
                       r_rows[b, pl.ds(48, 16)] * d_rows[b, pl.ds(48, 16)])
                cum = plsc.cumsum(acc)
                plsc.store_scatter(out_v, [jnp.full((16,), b, jnp.int32)],
                                   cum, mask=last_lane)

    pltpu.sync_copy(out_v, out_hbm.at[pl.ds(base, BW)])


@functools.partial(jax.jit, static_argnames=())
def _run(rs, ds, regions_weight, days_weight):
    mesh = plsc.VectorSubcoreMesh(core_axis_name="c", subcore_axis_name="s")
    f = functools.partial(
        pl.kernel,
        out_type=jax.ShapeDtypeStruct((B,), jnp.float32),
        mesh=mesh,
        scratch_types=[
            pltpu.VMEM((BW,), jnp.int32),
            pltpu.VMEM((BW,), jnp.int32),
            pltpu.VMEM((BW, D), jnp.float32),
            pltpu.VMEM((BW, D), jnp.float32),
            pltpu.VMEM((BW,), jnp.float32),
            pltpu.VMEM_SHARED((1000, D), jnp.float32),
            pltpu.VMEM_SHARED((1000, D), jnp.float32),
            pltpu.SemaphoreType.DMA((10,)),
        ],
        compiler_params=pltpu.CompilerParams(
            needs_layout_passes=False, use_tc_tiling_on_sc=False),
    )(_sc_body)
    return f(rs, ds, regions_weight, days_weight)


def kernel(rs, ds, regions_weight, days_weight):
    return _run(rs.astype(jnp.int32), ds.astype(jnp.int32),
                regions_weight, days_weight)


# speedup vs baseline: 1.0373x; 1.0373x over previous
import functools

import jax
import jax.numpy as jnp
from jax import lax
from jax.experimental import pallas as pl
from jax.experimental.pallas import tpu as pltpu
from jax.experimental.pallas import tpu_sc as plsc

B = 16384
D = 64
NC = 2
NS = 16
NW = NC * NS
BW = B // NW


def _sc_body(rs_hbm, ds_hbm, rw_hbm, dw_hbm, out_hbm,
             rs_v, ds_v, r_rows, d_rows, out_v, r_spm, d_spm, sem):
    wid = lax.axis_index("s") * NC + lax.axis_index("c")
    base = wid * BW

    icp0 = pltpu.async_copy(rs_hbm.at[pl.ds(base, BW)], rs_v, sem.at[8])
    icp1 = pltpu.async_copy(ds_hbm.at[pl.ds(base, BW)], ds_v, sem.at[9])

    sid = lax.axis_index("s")
    row0 = sid * 64

    @pl.when(sid < 15)
    def _stage_main():
        pltpu.sync_copy(rw_hbm.at[pl.ds(row0, 64)], r_spm.at[pl.ds(row0, 64)])
        pltpu.sync_copy(dw_hbm.at[pl.ds(row0, 64)], d_spm.at[pl.ds(row0, 64)])

    @pl.when(sid == 15)
    def _stage_tail():
        pltpu.sync_copy(rw_hbm.at[pl.ds(960, 40)], r_spm.at[pl.ds(960, 40)])
        pltpu.sync_copy(dw_hbm.at[pl.ds(960, 40)], d_spm.at[pl.ds(960, 40)])

    plsc.subcore_barrier()

    icp0.wait()
    icp1.wait()

    NPIPE = 2
    HALF = BW // NPIPE
    cps = []
    for h in range(NPIPE):
        sl = pl.ds(h * HALF, HALF)
        cps.append((
            pltpu.async_copy(r_spm.at[rs_v.at[sl]], r_rows.at[sl],
                             sem.at[2 * h]),
            pltpu.async_copy(d_spm.at[ds_v.at[sl]], d_rows.at[sl],
                             sem.at[2 * h + 1]),
        ))

    lane = lax.iota(jnp.int32, 16)
    last_lane = lane == 15

    for h in range(NPIPE):
        cps[h][0].wait()
        cps[h][1].wait()

        @plsc.parallel_loop(h * HALF, (h + 1) * HALF, 4, unroll=2)
        def block_body(b0):
            for u in range(4):
                b = b0 + u
                acc = (r_rows[b, pl.ds(0, 16)] * d_rows[b, pl.ds(0, 16)] +
                       r_rows[b, pl.ds(16, 16)] * d_rows[b, pl.ds(16, 16)]) + (
                       r_rows[b, pl.ds(32, 16)] * d_rows[b, pl.ds(32, 16)] +
                       r_rows[b, pl.ds(48, 16)] * d_rows[b, pl.ds(48, 16)])
                cum = plsc.cumsum(acc)
                plsc.store_scatter(out_v, [jnp.full((16,), b, jnp.int32)],
                                   cum, mask=last_lane)

    pltpu.sync_copy(out_v, out_hbm.at[pl.ds(base, BW)])


@functools.partial(jax.jit, static_argnames=())
def _run(rs, ds, regions_weight, days_weight):
    mesh = plsc.VectorSubcoreMesh(core_axis_name="c", subcore_axis_name="s")
    f = functools.partial(
        pl.kernel,
        out_type=jax.ShapeDtypeStruct((B,), jnp.float32),
        mesh=mesh,
        scratch_types=[
            pltpu.VMEM((BW,), jnp.int32),
            pltpu.VMEM((BW,), jnp.int32),
            pltpu.VMEM((BW, D), jnp.float32),
            pltpu.VMEM((BW, D), jnp.float32),
            pltpu.VMEM((BW,), jnp.float32),
            pltpu.VMEM_SHARED((1000, D), jnp.float32),
            pltpu.VMEM_SHARED((1000, D), jnp.float32),
            pltpu.SemaphoreType.DMA((10,)),
        ],
        compiler_params=pltpu.CompilerParams(
            needs_layout_passes=False, use_tc_tiling_on_sc=False),
    )(_sc_body)
    return f(rs, ds, regions_weight, days_weight)


def kernel(rs, ds, regions_weight, days_weight):
    return _run(rs.astype(jnp.int32), ds.astype(jnp.int32),
                regions_weight, days_weight)
